# baseline (speedup 1.0000x reference)
import jax
import jax.numpy as jnp
from jax.experimental import pallas as pl
from jax.experimental.pallas import tpu as pltpu

B, S, H = 16, 4096, 512


def _body(len_ref, seq_ref, out_ref, sem):
    for b in range(B):
        row = (len_ref[b] - 1) & (S - 1)
        pltpu.make_async_copy(seq_ref.at[b, row], out_ref.at[b], sem).start()
    pltpu.make_async_copy(seq_ref.at[0, pl.ds(0, B)], out_ref, sem).wait()


def kernel(sequence, lengths):
    return pl.pallas_call(
        _body,
        out_shape=jax.ShapeDtypeStruct((B, H), jnp.float32),
        in_specs=[
            pl.BlockSpec(memory_space=pltpu.MemorySpace.SMEM),
            pl.BlockSpec(memory_space=pl.ANY),
        ],
        scratch_shapes=[pltpu.SemaphoreType.DMA],
    )(lengths.astype(jnp.int32), sequence)
